# Initial kernel scaffold; baseline (speedup 1.0000x reference)
import functools

import jax
import jax.numpy as jnp
from jax import lax
from jax.experimental import pallas as pl
from jax.experimental.pallas import tpu as pltpu
from jax.experimental.pallas import tpu_sc as plsc

_B = 8
_N = 100000
_H = 256
_W = 256
_NBAND = 4
_ROWS = _H // _NBAND
_CH = 4000
_CH3 = _CH * 3
_NCHUNK = _N // _CH
_NGROUP = _CH // 16

_MAGIC = jnp.float32(2.0 ** 23)


def _sc_body(s_hbm, out_hbm, buf_v, grid_v):
    c = lax.axis_index("c")
    s = lax.axis_index("s")
    w = c * 16 + s
    b = w // _NBAND
    q = w % _NBAND
    rlo = q * _ROWS

    zeros16 = jnp.zeros((16,), jnp.float32)
    ones16 = jnp.ones((16,), jnp.float32)
    lane3 = lax.iota(jnp.int32, 16) * 3

    def zrow(r, carry):
        for cidx in range(_W // 16):
            grid_v[r, pl.ds(cidx * 16, 16)] = zeros16
        return carry

    lax.fori_loop(0, _ROWS, zrow, 0)

    def chunk_body(ci, carry):
        pltpu.sync_copy(s_hbm.at[b, pl.ds(ci * _CH3, _CH3)], buf_v)

        def grp(g, carry2):
            idx = lane3 + g * 48
            xi = plsc.load_gather(buf_v, [idx])
            yi = plsc.load_gather(buf_v, [idx + 1])
            rx = (xi * jnp.float32(255.0) + _MAGIC) - _MAGIC
            ry = (yi * jnp.float32(255.0) + _MAGIC) - _MAGIC
            ix = jnp.minimum(jnp.maximum(rx.astype(jnp.int32), 0), _H - 1)
            iy = jnp.minimum(jnp.maximum(ry.astype(jnp.int32), 0), _W - 1)
            m = (ix >= rlo) & (ix < rlo + _ROWS)
            plsc.store_scatter(grid_v, [ix - rlo, iy], ones16, m)
            return carry2

        lax.fori_loop(0, _NGROUP, grp, 0)
        return carry

    lax.fori_loop(0, _NCHUNK, chunk_body, 0)

    pltpu.sync_copy(grid_v, out_hbm.at[b, pl.ds(rlo, _ROWS)])


def kernel(s_cur, action):
    del action
    s_flat = s_cur.reshape(_B, _N * 3)
    mesh = plsc.VectorSubcoreMesh(core_axis_name="c", subcore_axis_name="s")
    occ = pl.kernel(
        _sc_body,
        mesh=mesh,
        out_type=jax.ShapeDtypeStruct((_B, _H, _W), jnp.float32),
        scratch_types=[
            pltpu.VMEM((_CH3,), jnp.float32),
            pltpu.VMEM((_ROWS, _W), jnp.float32),
        ],
    )(s_flat)
    return occ


# speedup vs baseline: 6.7313x; 6.7313x over previous
import jax
import jax.numpy as jnp
from jax import lax
from jax.experimental import pallas as pl
from jax.experimental.pallas import tpu as pltpu
from jax.experimental.pallas import tpu_sc as plsc

_B = 8
_N = 100000
_NPAD = 100096
_H = 256
_W = 256
_NBAND = 4
_ROWS = _H // _NBAND
_CH = 1664
_NCHUNK = 60
_EPI = 256
_NGRP = _CH // 16
_UNROLL = 4
_NEPI = 10

_MAGIC = 8388608.0


def _sc_body(xs_hbm, ys_hbm, out_hbm, bufx_v, bufy_v, grid_v):
    c = lax.axis_index("c")
    s = lax.axis_index("s")
    w = c * 16 + s
    b = w // _NBAND
    q = w % _NBAND
    rlo = q * _ROWS

    zeros16 = jnp.zeros((16,), jnp.float32)
    ones16 = jnp.ones((16,), jnp.float32)

    def zrow(r, carry):
        for k in range(_W // 16):
            grid_v[r, pl.ds(k * 16, 16)] = zeros16
        return carry

    lax.fori_loop(0, _ROWS, zrow, 0)

    def do_group(off, m):
        xv = bufx_v[b, pl.ds(off, 16)]
        yv = bufy_v[b, pl.ds(off, 16)]
        rx = (xv * 255.0 + _MAGIC) - _MAGIC
        ry = (yv * 255.0 + _MAGIC) - _MAGIC
        ix = rx.astype(jnp.int32)
        iy = ry.astype(jnp.int32)
        keep = (ix >= rlo) & (ix < rlo + _ROWS)
        keep = keep if m is None else (keep & m)
        plsc.store_scatter(grid_v, [ix - rlo, iy], ones16, mask=keep)

    def chunk_body(ci, carry):
        col = ci * _CH
        pltpu.sync_copy(xs_hbm.at[pl.ds(0, _B), pl.ds(col, _CH)], bufx_v)
        pltpu.sync_copy(ys_hbm.at[pl.ds(0, _B), pl.ds(col, _CH)], bufy_v)

        def grp(g, carry2):
            for u in range(_UNROLL):
                do_group((g * _UNROLL + u) * 16, None)
            return carry2

        lax.fori_loop(0, _NGRP // _UNROLL, grp, 0)
        return carry

    lax.fori_loop(0, _NCHUNK, chunk_body, 0)

    pltpu.sync_copy(xs_hbm.at[pl.ds(0, _B), pl.ds(_NCHUNK * _CH, _EPI)],
                    bufx_v.at[pl.ds(0, _B), pl.ds(0, _EPI)])
    pltpu.sync_copy(ys_hbm.at[pl.ds(0, _B), pl.ds(_NCHUNK * _CH, _EPI)],
                    bufy_v.at[pl.ds(0, _B), pl.ds(0, _EPI)])
    for g in range(_NEPI):
        do_group(g * 16, None)

    pltpu.sync_copy(grid_v, out_hbm.at[b, pl.ds(rlo, _ROWS)])


def kernel(s_cur, action):
    del action
    xs = jnp.pad(s_cur[:, :, 0], ((0, 0), (0, _NPAD - _N)))
    ys = jnp.pad(s_cur[:, :, 1], ((0, 0), (0, _NPAD - _N)))
    mesh = plsc.VectorSubcoreMesh(core_axis_name="c", subcore_axis_name="s")
    occ = pl.kernel(
        _sc_body,
        mesh=mesh,
        compiler_params=pltpu.CompilerParams(needs_layout_passes=False),
        out_type=jax.ShapeDtypeStruct((_B, _H, _W), jnp.float32),
        scratch_types=[
            pltpu.VMEM((_B, _CH), jnp.float32),
            pltpu.VMEM((_B, _CH), jnp.float32),
            pltpu.VMEM((_ROWS, _W), jnp.float32),
        ],
    )(xs, ys)
    return occ


# speedup vs baseline: 32.3373x; 4.8040x over previous
import jax
import jax.numpy as jnp
from jax import lax
from jax.experimental import pallas as pl
from jax.experimental.pallas import tpu as pltpu
from jax.experimental.pallas import tpu_sc as plsc

_B = 8
_N = 100000
_NPAD = 100096
_H = 256
_W = 256
_RANGE = 24960
_CH = 640
_NCHUNK = _RANGE // _CH
_NSLOT = 4
_NWAVE = _NCHUNK // _NSLOT
_EPI = 256
_NGRP = _CH // 16
_NEPI = 10

_MAGIC = 8388608.0


def _sc_body(xy_hbm, out_hbm, buf_v, grid_v, shared_s, sem0, sem1, sem2, sem3):
    c = lax.axis_index("c")
    s = lax.axis_index("s")
    lb = s // 4
    q = s % 4
    b = c * 4 + lb
    start = q * _RANGE
    sems = (sem0, sem1, sem2, sem3)

    zeros16 = jnp.zeros((16,), jnp.float32)
    ones16 = jnp.ones((16,), jnp.float32)

    def chunk_copy(ci, slot, ncol=_CH):
        return pltpu.make_async_copy(
            xy_hbm.at[pl.ds(0, 16), pl.ds(start + ci * _CH, ncol)],
            buf_v.at[slot, pl.ds(0, 16), pl.ds(0, ncol)], sems[slot])

    for k in range(_NSLOT):
        chunk_copy(k, k).start()

    def zrow(r, carry):
        for k in range(_W // 16):
            grid_v[r, pl.ds(k * 16, 16)] = zeros16
        return carry

    lax.fori_loop(0, _H, zrow, 0)

    def do_group(slot, off):
        xv = buf_v[slot, b, pl.ds(off, 16)]
        yv = buf_v[slot, 8 + b, pl.ds(off, 16)]
        rx = (xv * 255.0 + _MAGIC) - _MAGIC
        ry = (yv * 255.0 + _MAGIC) - _MAGIC
        ix = rx.astype(jnp.int32)
        iy = ry.astype(jnp.int32)
        plsc.store_scatter(grid_v, [ix, iy], ones16)

    def do_chunk(ci, slot):
        chunk_copy(ci, slot).wait()

        @plsc.parallel_loop(0, _NGRP, unroll=4)
        def _grp(g):
            do_group(slot, g * 16)

    def wave_body(w, carry):
        for k in range(_NSLOT):
            ci = w * _NSLOT + k
            do_chunk(ci, k)

            @pl.when(ci + _NSLOT < _NCHUNK)
            def _refill(k=k, ci=ci):
                chunk_copy(ci + _NSLOT, k).start()

        return carry

    lax.fori_loop(0, _NWAVE, wave_body, 0)
    for k in range(_NCHUNK - _NWAVE * _NSLOT):
        do_chunk(_NWAVE * _NSLOT + k, k)

    @pl.when(q == 3)
    def _epi():
        chunk_copy(_NCHUNK, 3, _EPI).start()
        chunk_copy(_NCHUNK, 3, _EPI).wait()
        for g in range(_NEPI):
            do_group(3, g * 16)

    for w in range(1, 4):
        jw = lax.rem(q + w, 4)
        pltpu.sync_copy(grid_v.at[pl.ds(jw * 64, 64)],
                        shared_s.at[pl.ds((lb * 4 + jw) * 64, 64)])
        plsc.subcore_barrier()
        pltpu.sync_copy(shared_s.at[pl.ds((lb * 4 + q) * 64, 64)],
                        grid_v.at[pl.ds(jw * 64, 64)])
        plsc.subcore_barrier()

    @plsc.parallel_loop(0, 64, unroll=2)
    def _acc(r):
        for k in range(_W // 16):
            cs = pl.ds(k * 16, 16)
            v = (grid_v[r, cs] + grid_v[64 + r, cs]
                 + grid_v[128 + r, cs] + grid_v[192 + r, cs])
            grid_v[r, cs] = jnp.minimum(v, 1.0)

    pltpu.sync_copy(grid_v.at[pl.ds(0, 64)], out_hbm.at[b, pl.ds(q * 64, 64)])


def kernel(s_cur, action):
    del action
    xy = jnp.concatenate([s_cur[:, :, 0], s_cur[:, :, 1]], axis=0)
    xy = jnp.pad(xy, ((0, 0), (0, _NPAD - _N)))
    mesh = plsc.VectorSubcoreMesh(core_axis_name="c", subcore_axis_name="s")
    occ = pl.kernel(
        _sc_body,
        mesh=mesh,
        compiler_params=pltpu.CompilerParams(needs_layout_passes=False),
        out_type=jax.ShapeDtypeStruct((_B, _H, _W), jnp.float32),
        scratch_types=[
            pltpu.VMEM((_NSLOT, 16, _CH), jnp.float32),
            pltpu.VMEM((_H, _W), jnp.float32),
            pltpu.VMEM_SHARED((1024, _W), jnp.float32),
            pltpu.SemaphoreType.DMA,
            pltpu.SemaphoreType.DMA,
            pltpu.SemaphoreType.DMA,
            pltpu.SemaphoreType.DMA,
        ],
    )(xy)
    return occ


# speedup vs baseline: 36.0827x; 1.1158x over previous
import jax
import jax.numpy as jnp
from jax import lax
from jax.experimental import pallas as pl
from jax.experimental.pallas import tpu as pltpu
from jax.experimental.pallas import tpu_sc as plsc

_B = 8
_N = 100000
_NPAD = 100096
_H = 256
_W = 256
_RANGE = 24960
_CH = 640
_NCHUNK = _RANGE // _CH
_NSLOT = 6
_NWAVE = _NCHUNK // _NSLOT
_EPI = 256
_NGRP = _CH // 16
_NEPI = 10

_MAGIC = 8388608.0


def _sc_body(xy_hbm, out_hbm, buf_v, grid_v, shared_s,
             sem0, sem1, sem2, sem3, sem4, sem5):
    c = lax.axis_index("c")
    s = lax.axis_index("s")
    lb = s // 4
    q = s % 4
    b = c * 4 + lb
    start = q * _RANGE
    sems = (sem0, sem1, sem2, sem3, sem4, sem5)

    zeros16 = jnp.zeros((16,), jnp.float32)
    ones16 = jnp.ones((16,), jnp.float32)

    def chunk_copy(ci, slot, ncol=_CH):
        return pltpu.make_async_copy(
            xy_hbm.at[pl.ds(c * 8, 8), pl.ds(start + ci * _CH, ncol)],
            buf_v.at[slot, pl.ds(0, 8), pl.ds(0, ncol)], sems[slot])

    for k in range(_NSLOT):
        chunk_copy(k, k).start()

    def zrow(r, carry):
        for k in range(_W // 16):
            grid_v[r, pl.ds(k * 16, 16)] = zeros16
        return carry

    lax.fori_loop(0, _H, zrow, 0)

    def do_group(slot, off):
        xv = buf_v[slot, lb, pl.ds(off, 16)]
        yv = buf_v[slot, 4 + lb, pl.ds(off, 16)]
        rx = (xv * 255.0 + _MAGIC) - _MAGIC
        ry = (yv * 255.0 + _MAGIC) - _MAGIC
        ix = rx.astype(jnp.int32)
        iy = ry.astype(jnp.int32)
        plsc.store_scatter(grid_v, [ix, iy], ones16)

    def do_chunk(ci, slot):
        chunk_copy(ci, slot).wait()

        @plsc.parallel_loop(0, _NGRP, unroll=4)
        def _grp(g):
            do_group(slot, g * 16)

    def wave_body(w, carry):
        for k in range(_NSLOT):
            ci = w * _NSLOT + k
            do_chunk(ci, k)

            @pl.when(ci + _NSLOT < _NCHUNK)
            def _refill(k=k, ci=ci):
                chunk_copy(ci + _NSLOT, k).start()

        return carry

    lax.fori_loop(0, _NWAVE, wave_body, 0)
    for k in range(_NCHUNK - _NWAVE * _NSLOT):
        do_chunk(_NWAVE * _NSLOT + k, k)

    @pl.when(q == 3)
    def _epi():
        chunk_copy(_NCHUNK, 5, _EPI).start()
        chunk_copy(_NCHUNK, 5, _EPI).wait()
        for g in range(_NEPI):
            do_group(5, g * 16)

    for w in range(1, 4):
        jw = lax.rem(q + w, 4)
        pltpu.sync_copy(grid_v.at[pl.ds(jw * 64, 64)],
                        shared_s.at[pl.ds((lb * 4 + jw) * 64, 64)])
        plsc.subcore_barrier()
        pltpu.sync_copy(shared_s.at[pl.ds((lb * 4 + q) * 64, 64)],
                        grid_v.at[pl.ds(jw * 64, 64)])
        plsc.subcore_barrier()

    @plsc.parallel_loop(0, 64, unroll=2)
    def _acc(r):
        for k in range(_W // 16):
            cs = pl.ds(k * 16, 16)
            v = (grid_v[r, cs] + grid_v[64 + r, cs]
                 + grid_v[128 + r, cs] + grid_v[192 + r, cs])
            grid_v[r, cs] = jnp.minimum(v, 1.0)

    pltpu.sync_copy(grid_v.at[pl.ds(0, 64)], out_hbm.at[b, pl.ds(q * 64, 64)])


def kernel(s_cur, action):
    del action
    xs, ys = s_cur[:, :, 0], s_cur[:, :, 1]
    xy = jnp.concatenate([xs[0:4], ys[0:4], xs[4:8], ys[4:8]], axis=0)
    xy = jnp.pad(xy, ((0, 0), (0, _NPAD - _N)))
    mesh = plsc.VectorSubcoreMesh(core_axis_name="c", subcore_axis_name="s")
    occ = pl.kernel(
        _sc_body,
        mesh=mesh,
        compiler_params=pltpu.CompilerParams(needs_layout_passes=False),
        out_type=jax.ShapeDtypeStruct((_B, _H, _W), jnp.float32),
        scratch_types=[
            pltpu.VMEM((_NSLOT, 8, _CH), jnp.float32),
            pltpu.VMEM((_H, _W), jnp.float32),
            pltpu.VMEM_SHARED((1024, _W), jnp.float32),
            pltpu.SemaphoreType.DMA,
            pltpu.SemaphoreType.DMA,
            pltpu.SemaphoreType.DMA,
            pltpu.SemaphoreType.DMA,
            pltpu.SemaphoreType.DMA,
            pltpu.SemaphoreType.DMA,
        ],
    )(xy)
    return occ


# speedup vs baseline: 37.8494x; 1.0490x over previous
import jax
import jax.numpy as jnp
from jax import lax
from jax.experimental import pallas as pl
from jax.experimental.pallas import tpu as pltpu
from jax.experimental.pallas import tpu_sc as plsc

_B = 8
_N = 100000
_NPB = 102400
_CPB = 12800
_H = 256
_W = 256
_TCOLS = 3200
_CH = 640
_NCH = _TCOLS // _CH
_NSLOT = 4
_NGRP = _CH // 16
_LIM3 = (40, 10, 0, 0, 0)

_MAGIC = 8388608.0


def _sc_body(xy_hbm, out_hbm, buf_v, grid_v, shared_s, sem0, sem1, sem2, sem3):
    s = lax.axis_index("s")
    c = lax.axis_index("c")
    lb = s // 4
    q = s % 4
    b = c * 4 + lb
    colbase = q * _TCOLS
    sems = (sem0, sem1, sem2, sem3)

    zeros16 = jnp.zeros((16,), jnp.float32)
    ones16 = jnp.ones((16,), jnp.float32)

    def chunk_copy(ci, slot):
        return pltpu.make_async_copy(
            xy_hbm.at[pl.ds(b * 16, 16), pl.ds(colbase + ci * _CH, _CH)],
            buf_v.at[slot, pl.ds(0, 16), pl.ds(0, _CH)], sems[slot])

    for k in range(_NSLOT):
        chunk_copy(k, k).start()

    def zrow(r, carry):
        for k in range(_W // 16):
            grid_v[r, pl.ds(k * 16, 16)] = zeros16
        return carry

    lax.fori_loop(0, _H, zrow, 0)

    def do_group(slot, r, off):
        xv = buf_v[slot, r, pl.ds(off, 16)]
        yv = buf_v[slot, 8 + r, pl.ds(off, 16)]
        rx = (xv * 255.0 + _MAGIC) - _MAGIC
        ry = (yv * 255.0 + _MAGIC) - _MAGIC
        ix = rx.astype(jnp.int32)
        iy = ry.astype(jnp.int32)
        plsc.store_scatter(grid_v, [ix, iy], ones16)

    for ci in range(_NCH):
        slot = ci % _NSLOT
        chunk_copy(ci, slot).wait()

        for r in range(7):
            @plsc.parallel_loop(0, _NGRP, unroll=4)
            def _grp(g, slot=slot, r=r):
                do_group(slot, r, g * 16)

        r7bound = jnp.where(q == 3, _LIM3[ci], _NGRP)

        @plsc.parallel_loop(0, r7bound, unroll=2)
        def _grp7(g, slot=slot):
            do_group(slot, 7, g * 16)

        if ci + _NSLOT < _NCH:
            chunk_copy(ci + _NSLOT, slot).start()

    for w in range(1, 4):
        jw = lax.rem(q + w, 4)
        pltpu.sync_copy(grid_v.at[pl.ds(jw * 64, 64)],
                        shared_s.at[pl.ds((lb * 4 + jw) * 64, 64)])
        plsc.subcore_barrier()
        pltpu.sync_copy(shared_s.at[pl.ds((lb * 4 + q) * 64, 64)],
                        grid_v.at[pl.ds(jw * 64, 64)])
        plsc.subcore_barrier()

    @plsc.parallel_loop(0, 64, unroll=2)
    def _acc(r):
        for k in range(_W // 16):
            cs = pl.ds(k * 16, 16)
            v = (grid_v[r, cs] + grid_v[64 + r, cs]
                 + grid_v[128 + r, cs] + grid_v[192 + r, cs])
            grid_v[r, cs] = jnp.minimum(v, 1.0)

    pltpu.sync_copy(grid_v.at[pl.ds(0, 64)], out_hbm.at[b, pl.ds(q * 64, 64)])


def kernel(s_cur, action):
    del action
    xs, ys = s_cur[:, :, 0], s_cur[:, :, 1]
    xp = jnp.pad(xs, ((0, 0), (0, _NPB - _N))).reshape(_B, 8, _CPB)
    yp = jnp.pad(ys, ((0, 0), (0, _NPB - _N))).reshape(_B, 8, _CPB)
    xy = jnp.concatenate([xp, yp], axis=1).reshape(_B * 16, _CPB)
    mesh = plsc.VectorSubcoreMesh(core_axis_name="c", subcore_axis_name="s")
    occ = pl.kernel(
        _sc_body,
        mesh=mesh,
        compiler_params=pltpu.CompilerParams(needs_layout_passes=False),
        out_type=jax.ShapeDtypeStruct((_B, _H, _W), jnp.float32),
        scratch_types=[
            pltpu.VMEM((_NSLOT, 16, _CH), jnp.float32),
            pltpu.VMEM((_H, _W), jnp.float32),
            pltpu.VMEM_SHARED((1024, _W), jnp.float32),
            pltpu.SemaphoreType.DMA,
            pltpu.SemaphoreType.DMA,
            pltpu.SemaphoreType.DMA,
            pltpu.SemaphoreType.DMA,
        ],
    )(xy)
    return occ
